# baseline (speedup 1.0000x reference)
import functools

import jax
import jax.numpy as jnp
from jax import lax
from jax.experimental import pallas as pl
from jax.experimental.pallas import tpu as pltpu
from jax.experimental.pallas import tpu_sc as plsc

VOCAB = 100000
EMB = 128
HID = 1024
SCALE = float(EMB) ** 0.5


_NW = 32
_CHUNK = 800


def _sc_gather_body(table_hbm, idx_hbm, out_hbm, idx_v, rows_v, sem,
                    *, n_tokens):
    b_per_w = n_tokens // _NW
    n_chunks = b_per_w // _CHUNK
    wid = lax.axis_index("s") * 2 + lax.axis_index("c")
    base = wid * b_per_w

    def chunk(i, carry):
        start = base + i * _CHUNK
        pltpu.sync_copy(idx_hbm.at[pl.ds(start, _CHUNK)], idx_v)
        pltpu.async_copy(table_hbm.at[idx_v], rows_v, sem).wait()
        pltpu.sync_copy(rows_v, out_hbm.at[pl.ds(start, _CHUNK)])
        return carry

    lax.fori_loop(0, n_chunks, chunk, 0)


def _sc_gather(table, idx_flat):
    n_tokens = idx_flat.shape[0]
    mesh = plsc.VectorSubcoreMesh(core_axis_name="c", subcore_axis_name="s")
    return pl.kernel(
        functools.partial(_sc_gather_body, n_tokens=n_tokens),
        out_type=jax.ShapeDtypeStruct((n_tokens, EMB), jnp.float32),
        mesh=mesh,
        scratch_types=[
            pltpu.VMEM((_CHUNK,), jnp.int32),
            pltpu.VMEM((_CHUNK, EMB), jnp.float32),
            pltpu.SemaphoreType.DMA,
        ],
    )(table, idx_flat)


_TOK_BLK = 2048
_NSLICE = 4


def _tc_project_body(emb_ref, ids_ref, w_ref, out_ref):
    mask = ids_ref[...] != 0
    e = jnp.where(mask, emb_ref[...], 0.0) * SCALE
    out_ref[...] = jnp.dot(e, w_ref[...], preferred_element_type=jnp.float32)


def _tc_body_chained(prev_ref, emb_ref, ids_ref, w_ref, out_ref):
    del prev_ref
    _tc_project_body(emb_ref, ids_ref, w_ref, out_ref)


def _tc_project_slice(prev, gathered_j, ids_col_j, w, blk_base, n_tokens):
    nblk = gathered_j.shape[0] // _TOK_BLK
    slice_specs = [
        pl.BlockSpec((_TOK_BLK, EMB), lambda i: (i, 0)),
        pl.BlockSpec((_TOK_BLK, 1), lambda i: (i, 0)),
        pl.BlockSpec((EMB, HID), lambda i: (0, 0)),
    ]
    out_spec = pl.BlockSpec((_TOK_BLK, HID), lambda i: (blk_base + i, 0))
    out_shape = jax.ShapeDtypeStruct((n_tokens, HID), jnp.float32)
    if prev is None:
        return pl.pallas_call(
            _tc_project_body,
            grid=(nblk,),
            in_specs=slice_specs,
            out_specs=out_spec,
            out_shape=out_shape,
        )(gathered_j, ids_col_j, w)
    return pl.pallas_call(
        _tc_body_chained,
        grid=(nblk,),
        in_specs=[pl.BlockSpec(memory_space=pl.ANY)] + slice_specs,
        out_specs=out_spec,
        out_shape=out_shape,
        input_output_aliases={0: 0},
    )(prev, gathered_j, ids_col_j, w)


def kernel(inputs, shared_weights, map_weights):
    b, l = inputs.shape
    n_tokens = b * l
    idx_flat = inputs.reshape(-1)
    sl = n_tokens // _NSLICE
    gathered = [
        _sc_gather(shared_weights, lax.slice(idx_flat, (j * sl,), ((j + 1) * sl,)))
        for j in range(_NSLICE)
    ]
    ids_col = idx_flat.reshape(-1, 1)
    out = None
    for j in range(_NSLICE):
        out = _tc_project_slice(
            out, gathered[j],
            lax.slice(ids_col, (j * sl, 0), ((j + 1) * sl, 1)),
            map_weights,
            blk_base=j * (sl // _TOK_BLK),
            n_tokens=n_tokens,
        )
    return out.reshape(b, l, HID)


# speedup vs baseline: 1.1404x; 1.1404x over previous
import functools

import jax
import jax.numpy as jnp
from jax import lax
from jax.experimental import pallas as pl
from jax.experimental.pallas import tpu as pltpu
from jax.experimental.pallas import tpu_sc as plsc

VOCAB = 100000
EMB = 128
HID = 1024
SCALE = float(EMB) ** 0.5


_NW = 32
_CHUNK = 400


def _sc_gather_body(table_hbm, idx_hbm, out_hbm, idx_a, idx_b, rows_a,
                    rows_b, sem_a, sem_b, *, n_tokens):
    b_per_w = n_tokens // _NW
    n_chunks = b_per_w // _CHUNK
    wid = lax.axis_index("s") * 2 + lax.axis_index("c")
    base = wid * b_per_w

    idx_bufs = [idx_a, idx_b]
    row_bufs = [rows_a, rows_b]
    sems = [sem_a, sem_b]
    copies = [None, None]

    pltpu.sync_copy(idx_hbm.at[pl.ds(base, _CHUNK)], idx_a)
    copies[0] = pltpu.async_copy(table_hbm.at[idx_a], rows_a, sem_a)
    for i in range(n_chunks):
        cur, nxt = i & 1, (i + 1) & 1
        if i + 1 < n_chunks:
            start = base + (i + 1) * _CHUNK
            pltpu.sync_copy(idx_hbm.at[pl.ds(start, _CHUNK)], idx_bufs[nxt])
            copies[nxt] = pltpu.async_copy(
                table_hbm.at[idx_bufs[nxt]], row_bufs[nxt], sems[nxt])
        copies[cur].wait()
        pltpu.sync_copy(row_bufs[cur], out_hbm.at[pl.ds(base + i * _CHUNK, _CHUNK)])


def _sc_gather(table, idx_flat):
    n_tokens = idx_flat.shape[0]
    mesh = plsc.VectorSubcoreMesh(core_axis_name="c", subcore_axis_name="s")
    return pl.kernel(
        functools.partial(_sc_gather_body, n_tokens=n_tokens),
        out_type=jax.ShapeDtypeStruct((n_tokens, EMB), jnp.float32),
        mesh=mesh,
        scratch_types=[
            pltpu.VMEM((_CHUNK,), jnp.int32),
            pltpu.VMEM((_CHUNK,), jnp.int32),
            pltpu.VMEM((_CHUNK, EMB), jnp.float32),
            pltpu.VMEM((_CHUNK, EMB), jnp.float32),
            pltpu.SemaphoreType.DMA,
            pltpu.SemaphoreType.DMA,
        ],
    )(table, idx_flat)


_TOK_BLK = 4096
_NSLICE = 1


def _tc_project_body(emb_ref, ids_ref, w_ref, out_ref):
    mask = ids_ref[...] != 0
    e = jnp.where(mask, emb_ref[...], 0.0) * SCALE
    out_ref[...] = jnp.dot(e, w_ref[...], preferred_element_type=jnp.float32)


def _tc_body_chained(prev_ref, emb_ref, ids_ref, w_ref, out_ref):
    del prev_ref
    _tc_project_body(emb_ref, ids_ref, w_ref, out_ref)


def _tc_project_slice(prev, gathered_j, ids_col_j, w, blk_base, n_tokens):
    nblk = gathered_j.shape[0] // _TOK_BLK
    slice_specs = [
        pl.BlockSpec((_TOK_BLK, EMB), lambda i: (i, 0)),
        pl.BlockSpec((_TOK_BLK, 1), lambda i: (i, 0)),
        pl.BlockSpec((EMB, HID), lambda i: (0, 0)),
    ]
    out_spec = pl.BlockSpec((_TOK_BLK, HID), lambda i: (blk_base + i, 0))
    out_shape = jax.ShapeDtypeStruct((n_tokens, HID), jnp.float32)
    if prev is None:
        return pl.pallas_call(
            _tc_project_body,
            grid=(nblk,),
            in_specs=slice_specs,
            out_specs=out_spec,
            out_shape=out_shape,
        )(gathered_j, ids_col_j, w)
    return pl.pallas_call(
        _tc_body_chained,
        grid=(nblk,),
        in_specs=[pl.BlockSpec(memory_space=pl.ANY)] + slice_specs,
        out_specs=out_spec,
        out_shape=out_shape,
        input_output_aliases={0: 0},
    )(prev, gathered_j, ids_col_j, w)


def kernel(inputs, shared_weights, map_weights):
    b, l = inputs.shape
    n_tokens = b * l
    idx_flat = inputs.reshape(-1)
    sl = n_tokens // _NSLICE
    gathered = [
        _sc_gather(shared_weights, lax.slice(idx_flat, (j * sl,), ((j + 1) * sl,)))
        for j in range(_NSLICE)
    ]
    ids_col = idx_flat.reshape(-1, 1)
    out = None
    for j in range(_NSLICE):
        out = _tc_project_slice(
            out, gathered[j],
            lax.slice(ids_col, (j * sl, 0), ((j + 1) * sl, 1)),
            map_weights,
            blk_base=j * (sl // _TOK_BLK),
            n_tokens=n_tokens,
        )
    return out.reshape(b, l, HID)
